# baseline (speedup 1.0000x reference)
import functools

import jax
import jax.numpy as jnp
from jax import lax
from jax.experimental import pallas as pl
from jax.experimental.pallas import tpu as pltpu
from jax.experimental.pallas import tpu_sc as plsc

B = 16384
TB = 2048
NW = 32
RPW = B // NW
NCH = RPW // 128


def _prep_body(se_ref, ge_ref, me_ref, Wsub_ref, bsub_ref, Wgr_ref, bgr_ref,
               W1_ref, b1_ref, ts_o, tg_o, tmb_o, wsub1_o, wgr1_o):
    f32 = jnp.float32
    W1 = W1_ref[...]
    ts_o[...] = jnp.dot(se_ref[...], W1[0:32, :], preferred_element_type=f32)
    tg_o[...] = jnp.dot(ge_ref[...], W1[32:64, :], preferred_element_type=f32)
    bias = (b1_ref[...]
            + jnp.dot(bsub_ref[...], W1[96:128, :], preferred_element_type=f32)
            + jnp.dot(bgr_ref[...], W1[128:160, :], preferred_element_type=f32))
    tmb_o[...] = jnp.dot(me_ref[...], W1[64:96, :], preferred_element_type=f32) + bias
    wsub1_o[...] = jnp.dot(Wsub_ref[...], W1[96:128, :], preferred_element_type=f32)
    wgr1_o[...] = jnp.dot(Wgr_ref[...], W1[128:160, :], preferred_element_type=f32)


def _prep(school_emb, goal_emb, method_emb, W_sub, b_sub, W_gr, b_gr, W1, b1):
    shp = lambda r: jax.ShapeDtypeStruct((r, 128), jnp.float32)
    return pl.pallas_call(
        _prep_body,
        out_shape=(shp(102), shp(22), shp(12), shp(15), shp(12)),
    )(school_emb, goal_emb, method_emb, W_sub, b_sub.reshape(1, 32),
      W_gr, b_gr.reshape(1, 32), W1, b1.reshape(1, 128))


CH = 128
NSET = 2


def _sc_body(si_hbm, gi_hbm, mi_hbm, ts_hbm, tg_hbm, tmb_hbm,
             e1_hbm, e2_hbm, e3_hbm,
             idx_s, idx_g, idx_m, bufs, gsem, wsem):
    cid = lax.axis_index("c")
    sid = lax.axis_index("s")
    wid = cid * 16 + sid
    gbase = wid * RPW
    grow = wid * NCH

    pltpu.sync_copy(si_hbm.at[pl.ds(grow, NCH)], idx_s)
    pltpu.sync_copy(gi_hbm.at[pl.ds(grow, NCH)], idx_g)
    pltpu.sync_copy(mi_hbm.at[pl.ds(grow, NCH)], idx_m)

    tbls = [ts_hbm, tg_hbm, tmb_hbm]
    idxs = [idx_s, idx_g, idx_m]
    outs = [e1_hbm, e2_hbm, e3_hbm]

    gh = [[None] * 3 for _ in range(NCH)]
    wh = [[None] * 3 for _ in range(NCH)]
    for p in range(NCH + 1):
        if p < NCH:
            d = p % NSET
            for k in range(3):
                if p >= NSET and wh[p - NSET][k] is not None:
                    wh[p - NSET][k].wait()
                    wh[p - NSET][k] = None
                gh[p][k] = pltpu.async_copy(
                    tbls[k].at[idxs[k].at[p]], bufs.at[d, k], gsem.at[d, k])
        q = p - 1
        if 0 <= q < NCH:
            d = q % NSET
            for k in range(3):
                gh[q][k].wait()
                wh[q][k] = pltpu.async_copy(
                    bufs.at[d, k], outs[k].at[pl.ds(gbase + q * CH, CH)],
                    wsem.at[d, k])

    for q in range(NCH):
        for k in range(3):
            if wh[q][k] is not None:
                wh[q][k].wait()


def _sc_gather(si2, gi2, mi2, ts, tg, tmb):
    mesh = plsc.VectorSubcoreMesh(core_axis_name="c", subcore_axis_name="s")
    e = jax.ShapeDtypeStruct((B, 128), jnp.float32)
    k = functools.partial(
        pl.kernel,
        mesh=mesh,
        out_type=(e, e, e),
        scratch_types=[
            pltpu.VMEM((NCH, 128), jnp.int32),
            pltpu.VMEM((NCH, 128), jnp.int32),
            pltpu.VMEM((NCH, 128), jnp.int32),
            pltpu.VMEM((NSET, 3, CH, 128), jnp.float32),
            pltpu.SemaphoreType.DMA((NSET, 3)),
            pltpu.SemaphoreType.DMA((NSET, 3)),
        ],
    )(_sc_body)
    return k(si2, gi2, mi2, ts, tg, tmb)


def _tail_body(e1_ref, e2_ref, e3_ref, subM_ref, grM_ref, wsub1_ref, wgr1_ref,
               W2_ref, b2_ref, W3_ref, b3_ref, out_ref):
    f32 = jnp.float32
    h1 = (e1_ref[...] + e2_ref[...] + e3_ref[...]
          + jnp.dot(subM_ref[...], wsub1_ref[...], preferred_element_type=f32)
          + jnp.dot(grM_ref[...], wgr1_ref[...], preferred_element_type=f32))
    h1 = jnp.maximum(h1, 0.0)
    h2 = jnp.maximum(jnp.dot(h1, W2_ref[...], preferred_element_type=f32) + b2_ref[...], 0.0)
    out_ref[...] = jnp.dot(h2, W3_ref[...], preferred_element_type=f32) + b3_ref[...]


def _tail(E1, E2, E3, subM, grM, wsub1, wgr1, W2, b2, W3, b3):
    nb = B // TB

    def batch_spec(w):
        return pl.BlockSpec((TB, w), lambda i: (i, 0))

    def full_spec(shape):
        return pl.BlockSpec(shape, lambda i: (0,) * len(shape))

    return pl.pallas_call(
        _tail_body,
        grid=(nb,),
        in_specs=[
            batch_spec(128), batch_spec(128), batch_spec(128),
            batch_spec(15), batch_spec(12),
            full_spec((15, 128)), full_spec((12, 128)),
            full_spec((128, 64)), full_spec((1, 64)),
            full_spec((64, 32)), full_spec((1, 32)),
        ],
        out_specs=pl.BlockSpec((TB, 32), lambda i: (i, 0)),
        out_shape=jax.ShapeDtypeStruct((B, 32), jnp.float32),
    )(E1, E2, E3, subM, grM, wsub1, wgr1, W2, b2.reshape(1, 64), W3,
      b3.reshape(1, 32))


def _sc_min_body(si_hbm, ts_hbm, out_hbm, idx_s, buf, sem):
    cid = lax.axis_index("c")
    sid = lax.axis_index("s")
    wid = cid * 16 + sid
    pltpu.sync_copy(si_hbm.at[pl.ds(wid, 1)], idx_s)
    pltpu.async_copy(ts_hbm.at[idx_s.at[0]], buf, sem).wait()
    pltpu.sync_copy(buf, out_hbm.at[pl.ds(wid * CH, CH)])


def _sc_min(si2, ts):
    mesh = plsc.VectorSubcoreMesh(core_axis_name="c", subcore_axis_name="s")
    k = functools.partial(
        pl.kernel,
        mesh=mesh,
        out_type=jax.ShapeDtypeStruct((NW * CH, 128), jnp.float32),
        scratch_types=[
            pltpu.VMEM((1, 128), jnp.int32),
            pltpu.VMEM((CH, 128), jnp.float32),
            pltpu.SemaphoreType.DMA,
        ],
    )(_sc_min_body)
    return k(si2, ts)


def kernel(school_idx, goal_idx, method_idx, subject_multi_hot, grade_multi_hot,
           school_emb, goal_emb, method_emb, W_sub, b_sub, W_gr, b_gr,
           W1, b1, W2, b2, W3, b3):
    ts, tg, tmb, wsub1, wgr1 = _prep(
        school_emb, goal_emb, method_emb, W_sub, b_sub, W_gr, b_gr, W1, b1)
    si2 = school_idx.astype(jnp.int32).reshape(B // 128, 128)
    gi2 = goal_idx.astype(jnp.int32).reshape(B // 128, 128)
    mi2 = method_idx.astype(jnp.int32).reshape(B // 128, 128)
    E1, E2, E3 = _sc_gather(si2, gi2, mi2, ts, tg, tmb)
    probe = _sc_min(si2, ts)
    out = _tail(E1, E2, E3, subject_multi_hot, grade_multi_hot,
                wsub1, wgr1, W2, b2, W3, b3)
    return out + 0.0 * probe[0, 0]


# speedup vs baseline: 1.4439x; 1.4439x over previous
import functools

import jax
import jax.numpy as jnp
from jax import lax
from jax.experimental import pallas as pl
from jax.experimental.pallas import tpu as pltpu
from jax.experimental.pallas import tpu_sc as plsc

B = 16384
TB = 2048
NW = 32
RPW = B // NW
NCH = RPW // 128


def _prep_body(se_ref, ge_ref, me_ref, Wsub_ref, bsub_ref, Wgr_ref, bgr_ref,
               W1_ref, b1_ref, ts_o, tg_o, tmb_o, wsub1_o, wgr1_o):
    f32 = jnp.float32
    W1 = W1_ref[...]
    ts_o[...] = jnp.dot(se_ref[...], W1[0:32, :], preferred_element_type=f32)
    tg_o[...] = jnp.dot(ge_ref[...], W1[32:64, :], preferred_element_type=f32)
    bias = (b1_ref[...]
            + jnp.dot(bsub_ref[...], W1[96:128, :], preferred_element_type=f32)
            + jnp.dot(bgr_ref[...], W1[128:160, :], preferred_element_type=f32))
    tmb_o[...] = jnp.dot(me_ref[...], W1[64:96, :], preferred_element_type=f32) + bias
    wsub1_o[...] = jnp.dot(Wsub_ref[...], W1[96:128, :], preferred_element_type=f32)
    wgr1_o[...] = jnp.dot(Wgr_ref[...], W1[128:160, :], preferred_element_type=f32)


def _prep(school_emb, goal_emb, method_emb, W_sub, b_sub, W_gr, b_gr, W1, b1):
    shp = lambda r: jax.ShapeDtypeStruct((r, 128), jnp.float32)
    return pl.pallas_call(
        _prep_body,
        out_shape=(shp(102), shp(22), shp(12), shp(15), shp(12)),
    )(school_emb, goal_emb, method_emb, W_sub, b_sub.reshape(1, 32),
      W_gr, b_gr.reshape(1, 32), W1, b1.reshape(1, 128))


CH = 128
NSET = 2


def _sc_body(si_hbm, gi_hbm, mi_hbm, ts_hbm, tg_hbm, tmb_hbm, out_hbm,
             idx_s, idx_g, idx_m, bufs, gsem, wsem):
    cid = lax.axis_index("c")
    sid = lax.axis_index("s")
    wid = cid * 16 + sid
    gbase = wid * RPW
    grow = wid * NCH

    pltpu.sync_copy(si_hbm.at[pl.ds(grow, NCH)], idx_s)
    pltpu.sync_copy(gi_hbm.at[pl.ds(grow, NCH)], idx_g)
    pltpu.sync_copy(mi_hbm.at[pl.ds(grow, NCH)], idx_m)

    def addpass(d):
        @plsc.parallel_loop(0, CH, step=1, unroll=4)
        def row_body(r):
            for k in range(8):
                sl = pl.ds(k * 16, 16)
                bufs[d, 2, r, sl] = (bufs[d, 2, r, sl]
                                     + bufs[d, 0, r, sl] + bufs[d, 1, r, sl])

    gh = [None] * NCH
    wh = [None] * NCH
    for p in range(NCH + NSET):
        q = p - NSET
        if 0 <= q < NCH:
            d = q % NSET
            for h in gh[q]:
                h.wait()
            addpass(d)
            wh[q] = pltpu.async_copy(
                bufs.at[d, 2], out_hbm.at[pl.ds(gbase + q * CH, CH)],
                wsem.at[d])
        if p < NCH:
            d = p % NSET
            g0 = pltpu.async_copy(ts_hbm.at[idx_s.at[p]], bufs.at[d, 0],
                                  gsem.at[d, 0])
            g1 = pltpu.async_copy(tg_hbm.at[idx_g.at[p]], bufs.at[d, 1],
                                  gsem.at[d, 1])
            if p >= NSET:
                wh[p - NSET].wait()
                wh[p - NSET] = None
            g2 = pltpu.async_copy(tmb_hbm.at[idx_m.at[p]], bufs.at[d, 2],
                                  gsem.at[d, 2])
            gh[p] = [g0, g1, g2]

    for q in range(NCH):
        if wh[q] is not None:
            wh[q].wait()


def _sc_gather(si2, gi2, mi2, ts, tg, tmb):
    mesh = plsc.VectorSubcoreMesh(core_axis_name="c", subcore_axis_name="s")
    k = functools.partial(
        pl.kernel,
        mesh=mesh,
        out_type=jax.ShapeDtypeStruct((B, 128), jnp.float32),
        scratch_types=[
            pltpu.VMEM((NCH, 128), jnp.int32),
            pltpu.VMEM((NCH, 128), jnp.int32),
            pltpu.VMEM((NCH, 128), jnp.int32),
            pltpu.VMEM((NSET, 3, CH, 128), jnp.float32),
            pltpu.SemaphoreType.DMA((NSET, 3)),
            pltpu.SemaphoreType.DMA((NSET,)),
        ],
    )(_sc_body)
    return k(si2, gi2, mi2, ts, tg, tmb)


def _tail_body(e_ref, subM_ref, grM_ref, wsub1_ref, wgr1_ref,
               W2_ref, b2_ref, W3_ref, b3_ref, out_ref):
    f32 = jnp.float32
    h1 = (e_ref[...]
          + jnp.dot(subM_ref[...], wsub1_ref[...], preferred_element_type=f32)
          + jnp.dot(grM_ref[...], wgr1_ref[...], preferred_element_type=f32))
    h1 = jnp.maximum(h1, 0.0)
    h2 = jnp.maximum(jnp.dot(h1, W2_ref[...], preferred_element_type=f32) + b2_ref[...], 0.0)
    out_ref[...] = jnp.dot(h2, W3_ref[...], preferred_element_type=f32) + b3_ref[...]


def _tail(E, subM, grM, wsub1, wgr1, W2, b2, W3, b3):
    nb = B // TB

    def batch_spec(w):
        return pl.BlockSpec((TB, w), lambda i: (i, 0))

    def full_spec(shape):
        return pl.BlockSpec(shape, lambda i: (0,) * len(shape))

    return pl.pallas_call(
        _tail_body,
        grid=(nb,),
        in_specs=[
            batch_spec(128), batch_spec(15), batch_spec(12),
            full_spec((15, 128)), full_spec((12, 128)),
            full_spec((128, 64)), full_spec((1, 64)),
            full_spec((64, 32)), full_spec((1, 32)),
        ],
        out_specs=pl.BlockSpec((TB, 32), lambda i: (i, 0)),
        out_shape=jax.ShapeDtypeStruct((B, 32), jnp.float32),
    )(E, subM, grM, wsub1, wgr1, W2, b2.reshape(1, 64), W3, b3.reshape(1, 32))


def kernel(school_idx, goal_idx, method_idx, subject_multi_hot, grade_multi_hot,
           school_emb, goal_emb, method_emb, W_sub, b_sub, W_gr, b_gr,
           W1, b1, W2, b2, W3, b3):
    ts, tg, tmb, wsub1, wgr1 = _prep(
        school_emb, goal_emb, method_emb, W_sub, b_sub, W_gr, b_gr, W1, b1)
    si2 = school_idx.astype(jnp.int32).reshape(B // 128, 128)
    gi2 = goal_idx.astype(jnp.int32).reshape(B // 128, 128)
    mi2 = method_idx.astype(jnp.int32).reshape(B // 128, 128)
    E = _sc_gather(si2, gi2, mi2, ts, tg, tmb)
    return _tail(E, subject_multi_hot, grade_multi_hot,
                 wsub1, wgr1, W2, b2, W3, b3)


# speedup vs baseline: 2.9182x; 2.0211x over previous
import functools

import jax
import jax.numpy as jnp
from jax import lax
from jax.experimental import pallas as pl
from jax.experimental.pallas import tpu as pltpu
from jax.experimental.pallas import tpu_sc as plsc

B = 16384
TB = 2048
NW = 32
RPW = B // NW
NCH = RPW // 128


NS_, NG_, NM_ = 102, 22, 12
NT3 = NS_ * NG_ * NM_


def _prep_body(se_ref, ge_ref, me_ref, Wsub_ref, bsub_ref, Wgr_ref, bgr_ref,
               W1_ref, b1_ref, si_ref, gi_ref, mi_ref,
               t3_o, wsub1_o, wgr1_o, ip_o):
    f32 = jnp.float32
    W1 = W1_ref[...]
    ts = jnp.dot(se_ref[...], W1[0:32, :], preferred_element_type=f32)
    tg = jnp.dot(ge_ref[...], W1[32:64, :], preferred_element_type=f32)
    bias = (b1_ref[...]
            + jnp.dot(bsub_ref[...], W1[96:128, :], preferred_element_type=f32)
            + jnp.dot(bgr_ref[...], W1[128:160, :], preferred_element_type=f32))
    tmb = jnp.dot(me_ref[...], W1[64:96, :], preferred_element_type=f32) + bias
    tsg = (ts[:, None, :] + tg[None, :, :]).reshape(NS_ * NG_, 128)
    t3_o[...] = (tsg[:, None, :] + tmb[None, :, :]).reshape(NT3, 128)
    wsub1_o[...] = jnp.dot(Wsub_ref[...], W1[96:128, :], preferred_element_type=f32)
    wgr1_o[...] = jnp.dot(Wgr_ref[...], W1[128:160, :], preferred_element_type=f32)
    ip_o[...] = (si_ref[...] * (NG_ * NM_) + gi_ref[...] * NM_ + mi_ref[...])


def _prep(school_emb, goal_emb, method_emb, W_sub, b_sub, W_gr, b_gr, W1, b1,
          si2, gi2, mi2):
    return pl.pallas_call(
        _prep_body,
        out_shape=(
            jax.ShapeDtypeStruct((NT3, 128), jnp.float32),
            jax.ShapeDtypeStruct((15, 128), jnp.float32),
            jax.ShapeDtypeStruct((12, 128), jnp.float32),
            jax.ShapeDtypeStruct((B // 128, 128), jnp.int32),
        ),
    )(school_emb, goal_emb, method_emb, W_sub, b_sub.reshape(1, 32),
      W_gr, b_gr.reshape(1, 32), W1, b1.reshape(1, 128), si2, gi2, mi2)


CH = 128
NSET = 2


def _sc_body(ip_hbm, t3_hbm, out_hbm, idx_p, bufs, gsem, wsem):
    cid = lax.axis_index("c")
    sid = lax.axis_index("s")
    wid = cid * 16 + sid
    gbase = wid * RPW
    grow = wid * NCH

    pltpu.sync_copy(ip_hbm.at[pl.ds(grow, NCH)], idx_p)

    gh = [None] * NCH
    wh = [None] * NCH
    for p in range(NCH + 1):
        if p < NCH:
            d = p % NSET
            if p >= NSET:
                wh[p - NSET].wait()
                wh[p - NSET] = None
            gh[p] = pltpu.async_copy(t3_hbm.at[idx_p.at[p]], bufs.at[d],
                                     gsem.at[d])
        q = p - 1
        if 0 <= q < NCH:
            d = q % NSET
            gh[q].wait()
            wh[q] = pltpu.async_copy(
                bufs.at[d], out_hbm.at[pl.ds(gbase + q * CH, CH)], wsem.at[d])

    for q in range(NCH):
        if wh[q] is not None:
            wh[q].wait()


def _sc_gather(ip2, t3):
    mesh = plsc.VectorSubcoreMesh(core_axis_name="c", subcore_axis_name="s")
    k = functools.partial(
        pl.kernel,
        mesh=mesh,
        out_type=jax.ShapeDtypeStruct((B, 128), jnp.float32),
        scratch_types=[
            pltpu.VMEM((NCH, 128), jnp.int32),
            pltpu.VMEM((NSET, CH, 128), jnp.float32),
            pltpu.SemaphoreType.DMA((NSET,)),
            pltpu.SemaphoreType.DMA((NSET,)),
        ],
    )(_sc_body)
    return k(ip2, t3)


def _tail_body(e_ref, subM_ref, grM_ref, wsub1_ref, wgr1_ref,
               W2_ref, b2_ref, W3_ref, b3_ref, out_ref):
    f32 = jnp.float32
    h1 = (e_ref[...]
          + jnp.dot(subM_ref[...], wsub1_ref[...], preferred_element_type=f32)
          + jnp.dot(grM_ref[...], wgr1_ref[...], preferred_element_type=f32))
    h1 = jnp.maximum(h1, 0.0)
    h2 = jnp.maximum(jnp.dot(h1, W2_ref[...], preferred_element_type=f32) + b2_ref[...], 0.0)
    out_ref[...] = jnp.dot(h2, W3_ref[...], preferred_element_type=f32) + b3_ref[...]


def _tail(E, subM, grM, wsub1, wgr1, W2, b2, W3, b3):
    nb = B // TB

    def batch_spec(w):
        return pl.BlockSpec((TB, w), lambda i: (i, 0))

    def full_spec(shape):
        return pl.BlockSpec(shape, lambda i: (0,) * len(shape))

    return pl.pallas_call(
        _tail_body,
        grid=(nb,),
        in_specs=[
            batch_spec(128), batch_spec(15), batch_spec(12),
            full_spec((15, 128)), full_spec((12, 128)),
            full_spec((128, 64)), full_spec((1, 64)),
            full_spec((64, 32)), full_spec((1, 32)),
        ],
        out_specs=pl.BlockSpec((TB, 32), lambda i: (i, 0)),
        out_shape=jax.ShapeDtypeStruct((B, 32), jnp.float32),
    )(E, subM, grM, wsub1, wgr1, W2, b2.reshape(1, 64), W3, b3.reshape(1, 32))


def kernel(school_idx, goal_idx, method_idx, subject_multi_hot, grade_multi_hot,
           school_emb, goal_emb, method_emb, W_sub, b_sub, W_gr, b_gr,
           W1, b1, W2, b2, W3, b3):
    si2 = school_idx.astype(jnp.int32).reshape(B // 128, 128)
    gi2 = goal_idx.astype(jnp.int32).reshape(B // 128, 128)
    mi2 = method_idx.astype(jnp.int32).reshape(B // 128, 128)
    t3, wsub1, wgr1, ip2 = _prep(
        school_emb, goal_emb, method_emb, W_sub, b_sub, W_gr, b_gr, W1, b1,
        si2, gi2, mi2)
    E = _sc_gather(ip2, t3)
    return _tail(E, subject_multi_hot, grade_multi_hot,
                 wsub1, wgr1, W2, b2, W3, b3)


# speedup vs baseline: 3.0036x; 1.0293x over previous
import functools

import jax
import jax.numpy as jnp
from jax import lax
from jax.experimental import pallas as pl
from jax.experimental.pallas import tpu as pltpu
from jax.experimental.pallas import tpu_sc as plsc

B = 16384
TB = 4096
NW = 32
RPW = B // NW
NCH = RPW // 128


NS_, NG_, NM_ = 102, 22, 12
NT3 = NS_ * NG_ * NM_


def _prep_body(se_ref, ge_ref, me_ref, Wsub_ref, bsub_ref, Wgr_ref, bgr_ref,
               W1_ref, b1_ref, si_ref, gi_ref, mi_ref,
               t3_o, wsub1_o, wgr1_o, ip_o):
    f32 = jnp.float32
    W1 = W1_ref[...]
    ts = jnp.dot(se_ref[...], W1[0:32, :], preferred_element_type=f32)
    tg = jnp.dot(ge_ref[...], W1[32:64, :], preferred_element_type=f32)
    bias = (b1_ref[...]
            + jnp.dot(bsub_ref[...], W1[96:128, :], preferred_element_type=f32)
            + jnp.dot(bgr_ref[...], W1[128:160, :], preferred_element_type=f32))
    tmb = jnp.dot(me_ref[...], W1[64:96, :], preferred_element_type=f32) + bias
    tsg = (ts[:, None, :] + tg[None, :, :]).reshape(NS_ * NG_, 128)
    t3_o[...] = (tsg[:, None, :] + tmb[None, :, :]).reshape(NT3, 128)
    wsub1_o[...] = jnp.dot(Wsub_ref[...], W1[96:128, :], preferred_element_type=f32)
    wgr1_o[...] = jnp.dot(Wgr_ref[...], W1[128:160, :], preferred_element_type=f32)
    ip_o[...] = (si_ref[...] * (NG_ * NM_) + gi_ref[...] * NM_ + mi_ref[...])


def _prep(school_emb, goal_emb, method_emb, W_sub, b_sub, W_gr, b_gr, W1, b1,
          si2, gi2, mi2):
    return pl.pallas_call(
        _prep_body,
        out_shape=(
            jax.ShapeDtypeStruct((NT3, 128), jnp.float32),
            jax.ShapeDtypeStruct((15, 128), jnp.float32),
            jax.ShapeDtypeStruct((12, 128), jnp.float32),
            jax.ShapeDtypeStruct((B // 128, 128), jnp.int32),
        ),
    )(school_emb, goal_emb, method_emb, W_sub, b_sub.reshape(1, 32),
      W_gr, b_gr.reshape(1, 32), W1, b1.reshape(1, 128), si2, gi2, mi2)


CH = 128
NSET = 4


def _sc_body(ip_hbm, t3_hbm, out_hbm, idx_p, bufs, gsem, wsem):
    cid = lax.axis_index("c")
    sid = lax.axis_index("s")
    wid = cid * 16 + sid
    gbase = wid * RPW
    grow = wid * NCH

    pltpu.sync_copy(ip_hbm.at[pl.ds(grow, NCH)], idx_p)

    gh = [None] * NCH
    wh = [None] * NCH
    for p in range(NCH + 1):
        if p < NCH:
            d = p % NSET
            if p >= NSET:
                wh[p - NSET].wait()
                wh[p - NSET] = None
            gh[p] = pltpu.async_copy(t3_hbm.at[idx_p.at[p]], bufs.at[d],
                                     gsem.at[d])
        q = p - 1
        if 0 <= q < NCH:
            d = q % NSET
            gh[q].wait()
            wh[q] = pltpu.async_copy(
                bufs.at[d], out_hbm.at[pl.ds(gbase + q * CH, CH)], wsem.at[d])

    for q in range(NCH):
        if wh[q] is not None:
            wh[q].wait()


def _sc_gather(ip2, t3):
    mesh = plsc.VectorSubcoreMesh(core_axis_name="c", subcore_axis_name="s")
    k = functools.partial(
        pl.kernel,
        mesh=mesh,
        out_type=jax.ShapeDtypeStruct((B, 128), jnp.float32),
        scratch_types=[
            pltpu.VMEM((NCH, 128), jnp.int32),
            pltpu.VMEM((NSET, CH, 128), jnp.float32),
            pltpu.SemaphoreType.DMA((NSET,)),
            pltpu.SemaphoreType.DMA((NSET,)),
        ],
    )(_sc_body)
    return k(ip2, t3)


def _tail_body(e_ref, subM_ref, grM_ref, wsub1_ref, wgr1_ref,
               W2_ref, b2_ref, W3_ref, b3_ref, out_ref):
    f32 = jnp.float32
    h1 = (e_ref[...]
          + jnp.dot(subM_ref[...], wsub1_ref[...], preferred_element_type=f32)
          + jnp.dot(grM_ref[...], wgr1_ref[...], preferred_element_type=f32))
    h1 = jnp.maximum(h1, 0.0)
    h2 = jnp.maximum(jnp.dot(h1, W2_ref[...], preferred_element_type=f32) + b2_ref[...], 0.0)
    out_ref[...] = jnp.dot(h2, W3_ref[...], preferred_element_type=f32) + b3_ref[...]


def _tail(E, subM, grM, wsub1, wgr1, W2, b2, W3, b3):
    nb = B // TB

    def batch_spec(w):
        return pl.BlockSpec((TB, w), lambda i: (i, 0))

    def full_spec(shape):
        return pl.BlockSpec(shape, lambda i: (0,) * len(shape))

    return pl.pallas_call(
        _tail_body,
        grid=(nb,),
        in_specs=[
            batch_spec(128), batch_spec(15), batch_spec(12),
            full_spec((15, 128)), full_spec((12, 128)),
            full_spec((128, 64)), full_spec((1, 64)),
            full_spec((64, 32)), full_spec((1, 32)),
        ],
        out_specs=pl.BlockSpec((TB, 32), lambda i: (i, 0)),
        out_shape=jax.ShapeDtypeStruct((B, 32), jnp.float32),
    )(E, subM, grM, wsub1, wgr1, W2, b2.reshape(1, 64), W3, b3.reshape(1, 32))


def kernel(school_idx, goal_idx, method_idx, subject_multi_hot, grade_multi_hot,
           school_emb, goal_emb, method_emb, W_sub, b_sub, W_gr, b_gr,
           W1, b1, W2, b2, W3, b3):
    si2 = school_idx.astype(jnp.int32).reshape(B // 128, 128)
    gi2 = goal_idx.astype(jnp.int32).reshape(B // 128, 128)
    mi2 = method_idx.astype(jnp.int32).reshape(B // 128, 128)
    t3, wsub1, wgr1, ip2 = _prep(
        school_emb, goal_emb, method_emb, W_sub, b_sub, W_gr, b_gr, W1, b1,
        si2, gi2, mi2)
    E = _sc_gather(ip2, t3)
    return _tail(E, subject_multi_hot, grade_multi_hot,
                 wsub1, wgr1, W2, b2, W3, b3)


# speedup vs baseline: 3.3852x; 1.1270x over previous
import functools

import jax
import jax.numpy as jnp
from jax import lax
from jax.experimental import pallas as pl
from jax.experimental.pallas import tpu as pltpu
from jax.experimental.pallas import tpu_sc as plsc

B = 16384
TB = 4096
NW = 32
RPW = B // NW
NCH = RPW // 128


NS_, NG_, NM_ = 102, 22, 12
NT2 = NS_ * NG_


def _prep_body(se_ref, ge_ref, me_ref, Wsub_ref, bsub_ref, Wgr_ref, bgr_ref,
               W1_ref, b1_ref, si_ref, gi_ref,
               t2_o, tmb_o, wsub1_o, wgr1_o, ip_o):
    f32 = jnp.float32
    W1 = W1_ref[...]
    ts = jnp.dot(se_ref[...], W1[0:32, :], preferred_element_type=f32)
    tg = jnp.dot(ge_ref[...], W1[32:64, :], preferred_element_type=f32)
    bias = (b1_ref[...]
            + jnp.dot(bsub_ref[...], W1[96:128, :], preferred_element_type=f32)
            + jnp.dot(bgr_ref[...], W1[128:160, :], preferred_element_type=f32))
    tmb_o[...] = jnp.dot(me_ref[...], W1[64:96, :], preferred_element_type=f32) + bias
    t2_o[...] = (ts[:, None, :] + tg[None, :, :]).reshape(NT2, 128)
    wsub1_o[...] = jnp.dot(Wsub_ref[...], W1[96:128, :], preferred_element_type=f32)
    wgr1_o[...] = jnp.dot(Wgr_ref[...], W1[128:160, :], preferred_element_type=f32)
    ip_o[...] = si_ref[...] * NG_ + gi_ref[...]


def _prep(school_emb, goal_emb, method_emb, W_sub, b_sub, W_gr, b_gr, W1, b1,
          si2, gi2):
    return pl.pallas_call(
        _prep_body,
        out_shape=(
            jax.ShapeDtypeStruct((NT2, 128), jnp.float32),
            jax.ShapeDtypeStruct((12, 128), jnp.float32),
            jax.ShapeDtypeStruct((15, 128), jnp.float32),
            jax.ShapeDtypeStruct((12, 128), jnp.float32),
            jax.ShapeDtypeStruct((B // 128, 128), jnp.int32),
        ),
    )(school_emb, goal_emb, method_emb, W_sub, b_sub.reshape(1, 32),
      W_gr, b_gr.reshape(1, 32), W1, b1.reshape(1, 128), si2, gi2)


CH = 128
NSET = 4


def _sc_body(ip_hbm, t3_hbm, out_hbm, idx_p, bufs, gsem, wsem):
    cid = lax.axis_index("c")
    sid = lax.axis_index("s")
    wid = cid * 16 + sid
    gbase = wid * RPW
    grow = wid * NCH

    pltpu.sync_copy(ip_hbm.at[pl.ds(grow, NCH)], idx_p)

    gh = [None] * NCH
    wh = [None] * NCH
    for p in range(NCH + 1):
        if p < NCH:
            d = p % NSET
            if p >= NSET:
                wh[p - NSET].wait()
                wh[p - NSET] = None
            gh[p] = pltpu.async_copy(t3_hbm.at[idx_p.at[p]], bufs.at[d],
                                     gsem.at[d])
        q = p - 1
        if 0 <= q < NCH:
            d = q % NSET
            gh[q].wait()
            wh[q] = pltpu.async_copy(
                bufs.at[d], out_hbm.at[pl.ds(gbase + q * CH, CH)], wsem.at[d])

    for q in range(NCH):
        if wh[q] is not None:
            wh[q].wait()


def _sc_gather(ip2, t3):
    mesh = plsc.VectorSubcoreMesh(core_axis_name="c", subcore_axis_name="s")
    k = functools.partial(
        pl.kernel,
        mesh=mesh,
        out_type=jax.ShapeDtypeStruct((B, 128), jnp.float32),
        scratch_types=[
            pltpu.VMEM((NCH, 128), jnp.int32),
            pltpu.VMEM((NSET, CH, 128), jnp.float32),
            pltpu.SemaphoreType.DMA((NSET,)),
            pltpu.SemaphoreType.DMA((NSET,)),
        ],
    )(_sc_body)
    return k(ip2, t3)


def _tail_body(e_ref, mi_ref, subM_ref, grM_ref, tmb_ref, wsub1_ref, wgr1_ref,
               W2_ref, b2_ref, W3_ref, b3_ref, out_ref):
    f32 = jnp.float32
    oh_m = (mi_ref[0, 0, :][:, None]
            == lax.broadcasted_iota(jnp.int32, (TB, 12), 1)).astype(f32)
    h1 = (e_ref[...]
          + jnp.dot(oh_m, tmb_ref[...], preferred_element_type=f32)
          + jnp.dot(subM_ref[...], wsub1_ref[...], preferred_element_type=f32)
          + jnp.dot(grM_ref[...], wgr1_ref[...], preferred_element_type=f32))
    h1 = jnp.maximum(h1, 0.0)
    h2 = jnp.maximum(jnp.dot(h1, W2_ref[...], preferred_element_type=f32) + b2_ref[...], 0.0)
    out_ref[...] = jnp.dot(h2, W3_ref[...], preferred_element_type=f32) + b3_ref[...]


def _tail(E, mi3, subM, grM, tmb, wsub1, wgr1, W2, b2, W3, b3):
    nb = B // TB

    def batch_spec(w):
        return pl.BlockSpec((TB, w), lambda i: (i, 0))

    def full_spec(shape):
        return pl.BlockSpec(shape, lambda i: (0,) * len(shape))

    return pl.pallas_call(
        _tail_body,
        grid=(nb,),
        in_specs=[
            batch_spec(128), pl.BlockSpec((1, 1, TB), lambda i: (i, 0, 0)),
            batch_spec(15), batch_spec(12),
            full_spec((12, 128)), full_spec((15, 128)), full_spec((12, 128)),
            full_spec((128, 64)), full_spec((1, 64)),
            full_spec((64, 32)), full_spec((1, 32)),
        ],
        out_specs=pl.BlockSpec((TB, 32), lambda i: (i, 0)),
        out_shape=jax.ShapeDtypeStruct((B, 32), jnp.float32),
    )(E, mi3, subM, grM, tmb, wsub1, wgr1, W2, b2.reshape(1, 64), W3,
      b3.reshape(1, 32))


def kernel(school_idx, goal_idx, method_idx, subject_multi_hot, grade_multi_hot,
           school_emb, goal_emb, method_emb, W_sub, b_sub, W_gr, b_gr,
           W1, b1, W2, b2, W3, b3):
    si2 = school_idx.astype(jnp.int32).reshape(B // 128, 128)
    gi2 = goal_idx.astype(jnp.int32).reshape(B // 128, 128)
    mi3 = method_idx.astype(jnp.int32).reshape(B // TB, 1, TB)
    t2, tmb, wsub1, wgr1, ip2 = _prep(
        school_emb, goal_emb, method_emb, W_sub, b_sub, W_gr, b_gr, W1, b1,
        si2, gi2)
    E = _sc_gather(ip2, t2)
    return _tail(E, mi3, subject_multi_hot, grade_multi_hot,
                 tmb, wsub1, wgr1, W2, b2, W3, b3)


# speedup vs baseline: 3.4312x; 1.0136x over previous
import functools

import jax
import jax.numpy as jnp
from jax import lax
from jax.experimental import pallas as pl
from jax.experimental.pallas import tpu as pltpu
from jax.experimental.pallas import tpu_sc as plsc

B = 16384
TB = 4096
NW = 32
RPW = B // NW
NCH = RPW // 128


NS_, NG_, NM_ = 102, 22, 12
NT2 = NS_ * NG_


def _prep_body(se_ref, ge_ref, me_ref, Wsub_ref, bsub_ref, Wgr_ref, bgr_ref,
               W1_ref, b1_ref, si_ref, gi_ref,
               t2_o, tmb_o, wsub1_o, wgr1_o, ip_o):
    f32 = jnp.float32
    W1 = W1_ref[...]
    ts = jnp.dot(se_ref[...], W1[0:32, :], preferred_element_type=f32)
    tg = jnp.dot(ge_ref[...], W1[32:64, :], preferred_element_type=f32)
    bias = (b1_ref[...]
            + jnp.dot(bsub_ref[...], W1[96:128, :], preferred_element_type=f32)
            + jnp.dot(bgr_ref[...], W1[128:160, :], preferred_element_type=f32))
    tmb_o[...] = jnp.dot(me_ref[...], W1[64:96, :], preferred_element_type=f32) + bias
    t2_o[...] = (ts[:, None, :] + tg[None, :, :]).reshape(NT2, 128)
    wsub1_o[...] = jnp.dot(Wsub_ref[...], W1[96:128, :], preferred_element_type=f32)
    wgr1_o[...] = jnp.dot(Wgr_ref[...], W1[128:160, :], preferred_element_type=f32)
    ip_o[...] = si_ref[...] * NG_ + gi_ref[...]


def _prep(school_emb, goal_emb, method_emb, W_sub, b_sub, W_gr, b_gr, W1, b1,
          si2, gi2):
    return pl.pallas_call(
        _prep_body,
        out_shape=(
            jax.ShapeDtypeStruct((NT2, 128), jnp.float32),
            jax.ShapeDtypeStruct((12, 128), jnp.float32),
            jax.ShapeDtypeStruct((15, 128), jnp.float32),
            jax.ShapeDtypeStruct((12, 128), jnp.float32),
            jax.ShapeDtypeStruct((B // 128, 128), jnp.int32),
        ),
    )(school_emb, goal_emb, method_emb, W_sub, b_sub.reshape(1, 32),
      W_gr, b_gr.reshape(1, 32), W1, b1.reshape(1, 128), si2, gi2)


CH = 128
NSET = 4


def _sc_body(ip_hbm, t3_hbm, out_hbm, idx_p, bufs, gsem, wsem):
    cid = lax.axis_index("c")
    sid = lax.axis_index("s")
    wid = cid * 16 + sid
    gbase = wid * RPW
    grow = wid * NCH

    pltpu.sync_copy(ip_hbm.at[pl.ds(grow, NCH)], idx_p)

    gh = [pltpu.async_copy(t3_hbm.at[idx_p.at[p]], bufs.at[p], gsem.at[p])
          for p in range(NCH)]
    wh = []
    for q in range(NCH):
        gh[q].wait()
        wh.append(pltpu.async_copy(
            bufs.at[q], out_hbm.at[pl.ds(gbase + q * CH, CH)], wsem.at[q]))
    for h in wh:
        h.wait()


def _sc_gather(ip2, t3):
    mesh = plsc.VectorSubcoreMesh(core_axis_name="c", subcore_axis_name="s")
    k = functools.partial(
        pl.kernel,
        mesh=mesh,
        out_type=jax.ShapeDtypeStruct((B, 128), jnp.float32),
        scratch_types=[
            pltpu.VMEM((NCH, 128), jnp.int32),
            pltpu.VMEM((NSET, CH, 128), jnp.float32),
            pltpu.SemaphoreType.DMA((NSET,)),
            pltpu.SemaphoreType.DMA((NSET,)),
        ],
    )(_sc_body)
    return k(ip2, t3)


def _tail_body(e_ref, mi_ref, subM_ref, grM_ref, tmb_ref, wsub1_ref, wgr1_ref,
               W2_ref, b2_ref, W3_ref, b3_ref, out_ref):
    f32 = jnp.float32
    oh_m = (mi_ref[0, 0, :][:, None]
            == lax.broadcasted_iota(jnp.int32, (TB, 12), 1)).astype(f32)
    h1 = (e_ref[...]
          + jnp.dot(oh_m, tmb_ref[...], preferred_element_type=f32)
          + jnp.dot(subM_ref[...], wsub1_ref[...], preferred_element_type=f32)
          + jnp.dot(grM_ref[...], wgr1_ref[...], preferred_element_type=f32))
    h1 = jnp.maximum(h1, 0.0)
    h2 = jnp.maximum(jnp.dot(h1, W2_ref[...], preferred_element_type=f32) + b2_ref[...], 0.0)
    out_ref[...] = jnp.dot(h2, W3_ref[...], preferred_element_type=f32) + b3_ref[...]


def _tail(E, mi3, subM, grM, tmb, wsub1, wgr1, W2, b2, W3, b3):
    nb = B // TB

    def batch_spec(w):
        return pl.BlockSpec((TB, w), lambda i: (i, 0))

    def full_spec(shape):
        return pl.BlockSpec(shape, lambda i: (0,) * len(shape))

    return pl.pallas_call(
        _tail_body,
        grid=(nb,),
        in_specs=[
            batch_spec(128), pl.BlockSpec((1, 1, TB), lambda i: (i, 0, 0)),
            batch_spec(15), batch_spec(12),
            full_spec((12, 128)), full_spec((15, 128)), full_spec((12, 128)),
            full_spec((128, 64)), full_spec((1, 64)),
            full_spec((64, 32)), full_spec((1, 32)),
        ],
        out_specs=pl.BlockSpec((TB, 32), lambda i: (i, 0)),
        out_shape=jax.ShapeDtypeStruct((B, 32), jnp.float32),
    )(E, mi3, subM, grM, tmb, wsub1, wgr1, W2, b2.reshape(1, 64), W3,
      b3.reshape(1, 32))


def kernel(school_idx, goal_idx, method_idx, subject_multi_hot, grade_multi_hot,
           school_emb, goal_emb, method_emb, W_sub, b_sub, W_gr, b_gr,
           W1, b1, W2, b2, W3, b3):
    si2 = school_idx.astype(jnp.int32).reshape(B // 128, 128)
    gi2 = goal_idx.astype(jnp.int32).reshape(B // 128, 128)
    mi3 = method_idx.astype(jnp.int32).reshape(B // TB, 1, TB)
    t2, tmb, wsub1, wgr1, ip2 = _prep(
        school_emb, goal_emb, method_emb, W_sub, b_sub, W_gr, b_gr, W1, b1,
        si2, gi2)
    E = _sc_gather(ip2, t2)
    return _tail(E, mi3, subject_multi_hot, grade_multi_hot,
                 tmb, wsub1, wgr1, W2, b2, W3, b3)
